# baseline (speedup 1.0000x reference)
import jax
import jax.numpy as jnp
from jax.experimental import pallas as pl

_TN = 2048


def _nnd_tile_kernel(x_ref, yt_ref, o1_ref, o2_ref):
    x = x_ref[0]
    yt = yt_ref[0]

    xk = [x[:, k:k + 1] for k in range(3)]
    yk = [yt[k:k + 1, :] for k in range(3)]

    x2 = (xk[0] * xk[0] + xk[1] * xk[1]) + xk[2] * xk[2]
    y2 = (yk[0] * yk[0] + yk[1] * yk[1]) + yk[2] * yk[2]

    xb = x.astype(jnp.bfloat16)
    yb2 = yt.astype(jnp.bfloat16) * jnp.bfloat16(-2.0)
    xy2 = jax.lax.dot_general(
        xb, yb2,
        (((1,), (0,)), ((), ())),
        preferred_element_type=jnp.float32,
    )

    d = (x2 + y2) + xy2

    tn = d.shape[0]
    rowmin = jnp.min(d, axis=1, keepdims=True)
    o1_ref[...] = rowmin.reshape(1, 1, tn)
    o2_ref[...] = jnp.min(d, axis=0, keepdims=True)[None]


def kernel(input1, input2):
    b, n, _ = input1.shape
    m = input2.shape[1]
    y_t = jnp.transpose(input2, (0, 2, 1))

    grid = (b, n // _TN)
    out1, out2 = pl.pallas_call(
        _nnd_tile_kernel,
        grid=grid,
        in_specs=[
            pl.BlockSpec((1, _TN, 3), lambda bi, ni: (bi, ni, 0)),
            pl.BlockSpec((1, 3, m), lambda bi, ni: (bi, 0, 0)),
        ],
        out_specs=[
            pl.BlockSpec((1, 1, _TN), lambda bi, ni: (bi, 0, ni)),
            pl.BlockSpec((1, 1, m), lambda bi, ni: (bi, 0, 0)),
        ],
        out_shape=[
            jax.ShapeDtypeStruct((b, 1, n), jnp.float32),
            jax.ShapeDtypeStruct((b, 1, m), jnp.float32),
        ],
    )(input1, y_t)

    return out1[:, 0, :], out2[:, 0, :]


# speedup vs baseline: 1.6353x; 1.6353x over previous
import jax
import jax.numpy as jnp
from jax.experimental import pallas as pl

_TN = 2048


def _nnd_tile_kernel(xt_ref, yt_ref, o1_ref, o2_ref):
    xt = xt_ref[0]
    yt = yt_ref[0]

    xk = [xt[k:k + 1, :] for k in range(3)]
    yk = [yt[k:k + 1, :] for k in range(3)]

    x2r = (xk[0] * xk[0] + xk[1] * xk[1]) + xk[2] * xk[2]
    y2 = (yk[0] * yk[0] + yk[1] * yk[1]) + yk[2] * yk[2]
    x2 = jnp.transpose(x2r)

    xb2 = xt.astype(jnp.bfloat16) * jnp.bfloat16(-2.0)
    yb = yt.astype(jnp.bfloat16)
    xy2 = jax.lax.dot_general(
        xb2, yb,
        (((0,), (0,)), ((), ())),
        preferred_element_type=jnp.float32,
    )

    d = (x2 + y2) + xy2

    tn = d.shape[0]
    o1_ref[...] = jnp.min(d, axis=1, keepdims=True).reshape(1, tn, 1)
    o2_ref[...] = jnp.min(d, axis=0, keepdims=True)[None]


def kernel(input1, input2):
    b, n, _ = input1.shape
    m = input2.shape[1]
    x_t = jnp.transpose(input1, (0, 2, 1))
    y_t = jnp.transpose(input2, (0, 2, 1))

    grid = (b, n // _TN)
    out1, out2 = pl.pallas_call(
        _nnd_tile_kernel,
        grid=grid,
        in_specs=[
            pl.BlockSpec((1, 3, _TN), lambda bi, ni: (bi, 0, ni)),
            pl.BlockSpec((1, 3, m), lambda bi, ni: (bi, 0, 0)),
        ],
        out_specs=[
            pl.BlockSpec((1, _TN, 1), lambda bi, ni: (bi, ni, 0)),
            pl.BlockSpec((1, 1, m), lambda bi, ni: (bi, 0, 0)),
        ],
        out_shape=[
            jax.ShapeDtypeStruct((b, n, 1), jnp.float32),
            jax.ShapeDtypeStruct((b, 1, m), jnp.float32),
        ],
    )(x_t, y_t)

    return out1[:, :, 0], out2[:, 0, :]


# speedup vs baseline: 1.7291x; 1.0574x over previous
import jax
import jax.numpy as jnp
from jax.experimental import pallas as pl

_TN = 2048


def _nnd_tile_kernel(xt_ref, yt_ref, o1_ref, o2_ref):
    xt = xt_ref[0]
    yt = yt_ref[0]

    xk = [xt[k:k + 1, :] for k in range(3)]
    yk = [yt[k:k + 1, :] for k in range(3)]

    x2r = (xk[0] * xk[0] + xk[1] * xk[1]) + xk[2] * xk[2]
    y2 = (yk[0] * yk[0] + yk[1] * yk[1]) + yk[2] * yk[2]
    x2 = jnp.transpose(x2r)

    xb2 = xt.astype(jnp.bfloat16) * jnp.bfloat16(-2.0)
    yb = yt.astype(jnp.bfloat16)
    xy2 = jax.lax.dot_general(
        xb2, yb,
        (((0,), (0,)), ((), ())),
        preferred_element_type=jnp.float32,
    )

    d = (x2 + y2) + xy2

    rowmin = jnp.min(d, axis=1, keepdims=True)
    o1_ref[...] = jnp.transpose(rowmin)[None]
    o2_ref[...] = jnp.min(d, axis=0, keepdims=True)[None]


def kernel(input1, input2):
    b, n, _ = input1.shape
    m = input2.shape[1]
    x_t = jnp.transpose(input1, (0, 2, 1))
    y_t = jnp.transpose(input2, (0, 2, 1))

    grid = (b, n // _TN)
    out1, out2 = pl.pallas_call(
        _nnd_tile_kernel,
        grid=grid,
        in_specs=[
            pl.BlockSpec((1, 3, _TN), lambda bi, ni: (bi, 0, ni)),
            pl.BlockSpec((1, 3, m), lambda bi, ni: (bi, 0, 0)),
        ],
        out_specs=[
            pl.BlockSpec((1, 1, _TN), lambda bi, ni: (bi, 0, ni)),
            pl.BlockSpec((1, 1, m), lambda bi, ni: (bi, 0, 0)),
        ],
        out_shape=[
            jax.ShapeDtypeStruct((b, 1, n), jnp.float32),
            jax.ShapeDtypeStruct((b, 1, m), jnp.float32),
        ],
    )(x_t, y_t)

    return out1[:, 0, :], out2[:, 0, :]
